# Initial kernel scaffold; baseline (speedup 1.0000x reference)
import jax
import jax.numpy as jnp
from jax.experimental import pallas as pl


def kernel(input_features, Wg, bg, We, be):
    raise NotImplementedError("write your pallas kernel here")


# speedup vs baseline: 1.3727x; 1.3727x over previous
import jax
import jax.numpy as jnp
from jax.experimental import pallas as pl
from jax.experimental.pallas import tpu as pltpu

_L = 4
_E = 8
_D = 80
_BLK = 512


def _top2_coeffs(g):
    T = g.shape[0]
    ids = jax.lax.broadcasted_iota(jnp.int32, (T, _E), 1)
    idf = ids.astype(jnp.float32)

    v0 = jnp.max(g, axis=1, keepdims=True)
    idx0 = jnp.min(jnp.where(g == v0, ids, _E), axis=1, keepdims=True)
    g1 = jnp.where(ids == idx0, -jnp.inf, g)
    v1 = jnp.max(g1, axis=1, keepdims=True)
    idx1 = jnp.min(jnp.where(g1 == v1, ids, _E), axis=1, keepdims=True)

    idx0f = idx0.astype(jnp.float32)
    idx1f = idx1.astype(jnp.float32)
    c = (jnp.where(v0 == idf, idx0f, 0.0)
         + jnp.where(v1 == idf, idx1f, 0.0))
    return c


def _moe_body(x_ref, wgt_ref, bg_ref, wet_ref, be_ref, o_ref):
    x = x_ref[...]
    T = x.shape[0]

    for layer in range(_L):
        g = (jnp.dot(x, wgt_ref[layer], preferred_element_type=jnp.float32)
             + bg_ref[layer][None, :])
        c = _top2_coeffs(g)
        active = jnp.any(c != 0.0)

        def dense(x, c):
            acc = jnp.zeros((T, _D), jnp.float32)
            for e in range(_E):
                eo = (jnp.dot(x, wet_ref[e], preferred_element_type=jnp.float32)
                      + be_ref[e][None, :])
                acc = acc + c[:, e:e + 1] * eo
            return acc

        def zero(x, c):
            return jnp.zeros((T, _D), jnp.float32)

        x = jax.lax.cond(active, dense, zero, x, c)

    o_ref[...] = x


def kernel(input_features, Wg, bg, We, be, interpret=False):
    B, S, D = input_features.shape
    N = B * S
    x = input_features.reshape(N, D)
    WgT = jnp.transpose(Wg, (0, 2, 1))
    WeT = jnp.transpose(We, (0, 2, 1))

    grid = (N // _BLK,)
    out = pl.pallas_call(
        _moe_body,
        grid=grid,
        in_specs=[
            pl.BlockSpec((_BLK, D), lambda i: (i, 0)),
            pl.BlockSpec((_L, D, _E), lambda i: (0, 0, 0)),
            pl.BlockSpec((_L, _E), lambda i: (0, 0)),
            pl.BlockSpec((_E, D, D), lambda i: (0, 0, 0)),
            pl.BlockSpec((_E, D), lambda i: (0, 0)),
        ],
        out_specs=pl.BlockSpec((_BLK, D), lambda i: (i, 0)),
        out_shape=jax.ShapeDtypeStruct((N, D), jnp.float32),
        interpret=interpret,
    )(x, WgT, bg, WeT, be)
    return out.reshape(B, S, D)
